# Initial kernel scaffold; baseline (speedup 1.0000x reference)
import jax
import jax.numpy as jnp
from jax.experimental import pallas as pl


def kernel(inputs, targets):
    raise NotImplementedError("write your pallas kernel here")


# speedup vs baseline: 14.3787x; 14.3787x over previous
import functools

import jax
import jax.numpy as jnp
from jax.experimental import pallas as pl

_M = 4096
_N = 16384
_DELTA = 5.0
_K = 163

_BR = 32
_CW = 256
_ITERS = 16


def _mmcl_body(x_ref, t_ref, o_ref):
    i = pl.program_id(0)
    nch = _N // _CW
    tgt = t_ref[...]
    col0 = jax.lax.broadcasted_iota(jnp.int32, (_BR, _CW), 1)
    kf = jnp.float32(_K)

    def p1(c, carry):
        mx, mn, ps = carry
        x = x_ref[:, pl.ds(c * _CW, _CW)]
        isp = col0 == (tgt - c * _CW)
        ps = ps + jnp.sum(jnp.where(isp, x, 0.0), axis=1, keepdims=True)
        mx = jnp.maximum(mx, jnp.max(x, axis=1, keepdims=True))
        mn = jnp.minimum(mn, jnp.min(x, axis=1, keepdims=True))
        return mx, mn, ps

    init = (jnp.full((_BR, 1), -jnp.inf, jnp.float32),
            jnp.full((_BR, 1), jnp.inf, jnp.float32),
            jnp.zeros((_BR, 1), jnp.float32))
    mx, mn, pos = jax.lax.fori_loop(0, nch, p1, init)

    def bis(j, carry):
        lo, hi = carry
        mid = 0.5 * lo + 0.5 * hi

        def cchunk(c, acc):
            x = x_ref[:, pl.ds(c * _CW, _CW)]
            return acc + (x >= mid).astype(jnp.float32)

        acc = jax.lax.fori_loop(0, nch, cchunk,
                                jnp.zeros((_BR, _CW), jnp.float32))
        cnt = (jnp.sum(acc, axis=1, keepdims=True)
               - (pos >= mid).astype(jnp.float32))
        ok = cnt >= kf
        return jnp.where(ok, mid, lo), jnp.where(ok, hi, mid)

    lo, _ = jax.lax.fori_loop(0, _ITERS, bis, (mn, mx))

    def p3(c, carry):
        s, cgt = carry
        x = x_ref[:, pl.ds(c * _CW, _CW)]
        ge = x >= lo
        v = 1.0 + x
        s = s + jnp.sum(jnp.where(ge, v * v, 0.0), axis=1, keepdims=True)
        cgt = cgt + jnp.sum(ge.astype(jnp.float32), axis=1, keepdims=True)
        return s, cgt

    s, cgt = jax.lax.fori_loop(
        0, nch, p3,
        (jnp.zeros((_BR, 1), jnp.float32), jnp.zeros((_BR, 1), jnp.float32)))
    posge = pos >= lo
    pv = 1.0 + pos
    s = s - jnp.where(posge, pv * pv, 0.0)
    cgt = cgt - posge.astype(jnp.float32)
    tlo = 1.0 + lo
    top = s - (cgt - kf) * (tlo * tlo)
    per_row = _DELTA * (1.0 - pos) ** 2 + top * (1.0 / kf)
    blk = jnp.sum(per_row) * (1.0 / _M)

    @pl.when(i == 0)
    def _init():
        o_ref[...] = jnp.zeros_like(o_ref)

    o_ref[...] += jnp.reshape(blk, (1, 1))


@functools.partial(jax.jit, static_argnames=())
def kernel(inputs, targets):
    t2 = targets.reshape(_M, 1).astype(jnp.int32)
    out = pl.pallas_call(
        _mmcl_body,
        grid=(_M // _BR,),
        in_specs=[
            pl.BlockSpec((_BR, _N), lambda i: (i, 0)),
            pl.BlockSpec((_BR, 1), lambda i: (i, 0)),
        ],
        out_specs=pl.BlockSpec((1, 1), lambda i: (0, 0)),
        out_shape=jax.ShapeDtypeStruct((1, 1), jnp.float32),
    )(inputs, t2)
    return out[0, 0]
